# baseline (speedup 1.0000x reference)
import functools

import jax
import jax.numpy as jnp
from jax import lax
from jax.experimental import pallas as pl
from jax.experimental.pallas import tpu as pltpu
from jax.experimental.pallas import tpu_sc as plsc

_D = 256
_ROWS = 32 * 1024
_NC = 2
_NS = 16
_NW = _NC * _NS
_RPW = _ROWS // _NW
_CH = 128
_NCH = _RPW // _CH


@functools.partial(
    pl.kernel,
    mesh=plsc.VectorSubcoreMesh(core_axis_name="c", subcore_axis_name="s"),
    out_type=jax.ShapeDtypeStruct((_ROWS, _D), jnp.float32),
    scratch_types=[
        pltpu.MemorySpace.VMEM_SHARED((_NS, _CH, _D), jnp.float32),
        pltpu.SemaphoreType.DMA((2,)),
    ],
)
def _sc_copy(x_hbm, o_hbm, buf, sems):
    sid = lax.axis_index("s")
    wid = sid * _NC + lax.axis_index("c")
    base = wid * _RPW
    for j in range(_NCH):
        pltpu.make_async_copy(
            x_hbm.at[pl.ds(base + j * _CH, _CH)], buf.at[sid], sems.at[0]).start()
        pltpu.make_async_copy(
            x_hbm.at[pl.ds(base + j * _CH, _CH)], buf.at[sid], sems.at[0]).wait()
        pltpu.make_async_copy(
            buf.at[sid], o_hbm.at[pl.ds(base + j * _CH, _CH)], sems.at[1]).start()
        pltpu.make_async_copy(
            buf.at[sid], o_hbm.at[pl.ds(base + j * _CH, _CH)], sems.at[1]).wait()


def kernel(x):
    return _sc_copy(x.reshape(-1, _D))


# speedup vs baseline: 2.0732x; 2.0732x over previous
import jax
import jax.numpy as jnp
from jax.experimental import pallas as pl

_D = 256
_BLOCK_ROWS = 8192


def _copy_body(x_ref, o_ref):
    o_ref[...] = x_ref[...]


def kernel(x):
    x2 = x.reshape(-1, _D)
    m = x2.shape[0]
    grid = m // _BLOCK_ROWS
    return pl.pallas_call(
        _copy_body,
        grid=(grid,),
        in_specs=[pl.BlockSpec((_BLOCK_ROWS, _D), lambda i: (i, 0))],
        out_specs=pl.BlockSpec((_BLOCK_ROWS, _D), lambda i: (i, 0)),
        out_shape=jax.ShapeDtypeStruct((m, _D), x2.dtype),
    )(x2)
